# baseline (speedup 1.0000x reference)
import functools

import jax
import jax.numpy as jnp
import numpy as np
from jax import lax
from jax.experimental import pallas as pl
from jax.experimental.pallas import tpu as pltpu
from jax.experimental.pallas import tpu_sc as plsc

D0 = 128
_NC = 2
_NS = 16
_NW = _NC * _NS

_BN = 256


def _pad256(n):
    return ((n + 255) // 256) * 256


def _keys(c, D):
    return ((c[:, 0] * D + c[:, 1]) * D + c[:, 2]) * D + c[:, 3]


def _subm_rb(coords, D):
    k = _keys(coords, D)
    order = jnp.argsort(k, stable=True)
    sk = k[order]
    n = coords.shape[0]
    rb = []
    for dz in (-1, 0, 1):
        for dy in (-1, 0, 1):
            for dx in (-1, 0, 1):
                q1 = coords[:, 1] + dz
                q2 = coords[:, 2] + dy
                q3 = coords[:, 3] + dx
                valid = ((q1 >= 0) & (q1 < D) & (q2 >= 0) & (q2 < D)
                         & (q3 >= 0) & (q3 < D))
                qk = jnp.where(valid, ((coords[:, 0] * D + q1) * D + q2) * D + q3, -1)
                pos = jnp.clip(jnp.searchsorted(sk, qk), 0, n - 1)
                found = (sk[pos] == qk) & valid
                rb.append((order[pos], found))
    return rb


def _down_rb(coords, cap):
    parent = jnp.concatenate([coords[:, :1], coords[:, 1:] // 2], axis=1)
    fill = jnp.array([1, 0, 0, 0], coords.dtype)
    oc, inv = jnp.unique(parent, axis=0, size=cap, fill_value=fill,
                         return_inverse=True)
    inv = inv.reshape(-1)
    kidx = (coords[:, 1] % 2) * 4 + (coords[:, 2] % 2) * 2 + (coords[:, 3] % 2)
    return oc, inv, kidx


_GK_CACHE = {}


def _make_gather(V, C, B):
    R = 128
    nchunks = B // R
    per_worker = -(-nchunks // _NW)
    mesh = plsc.VectorSubcoreMesh(core_axis_name="c", subcore_axis_name="s")

    @functools.partial(
        pl.kernel, mesh=mesh,
        out_type=jax.ShapeDtypeStruct((B, C), jnp.float32),
        compiler_params=pltpu.CompilerParams(use_tc_tiling_on_sc=False),
        scratch_types=[
            pltpu.VMEM((R,), jnp.int32),
            pltpu.VMEM((R, C), jnp.float32),
            pltpu.SemaphoreType.DMA,
        ],
    )
    def gk(table_h, idx_h, out_h, idx_v, rows_v, sem):
        wid = lax.axis_index("s") * _NC + lax.axis_index("c")

        def body(j, carry):
            chunk = wid + j * _NW

            @pl.when(chunk < nchunks)
            def _():
                base = chunk * R
                pltpu.sync_copy(idx_h.at[pl.ds(base, R)], idx_v)
                pltpu.async_copy(table_h.at[idx_v], rows_v, sem).wait()
                pltpu.sync_copy(rows_v, out_h.at[pl.ds(base, R)])

            return carry

        lax.fori_loop(0, per_worker, body, 0)

    return gk


def _gather_rows(table, idx):
    V, C = table.shape
    B = idx.shape[0]
    key = (V, C, B)
    if key not in _GK_CACHE:
        _GK_CACHE[key] = _make_gather(V, C, B)
    return _GK_CACHE[key](table, idx)


def _dot(a, b):
    return lax.dot_general(a, b, (((1,), (0,)), ((), ())),
                           precision=lax.Precision.HIGHEST,
                           preferred_element_type=jnp.float32)


def _tapmm(G, W, b, res, relu, nvalid):
    Bflat, cin = G.shape
    T, _, cout = W.shape
    NP = Bflat // T
    nb = NP // _BN
    b8 = jnp.broadcast_to(b.reshape(1, cout), (8, cout))

    in_specs = [
        pl.BlockSpec((_BN, cin), lambda i, t: (t * nb + i, 0)),
        pl.BlockSpec((1, cin, cout), lambda i, t: (t, 0, 0)),
        pl.BlockSpec((8, cout), lambda i, t: (0, 0)),
    ]
    args = [G, W, b8]
    if res is not None:
        in_specs.append(pl.BlockSpec((_BN, cout), lambda i, t: (i, 0)))
        args.append(res)

    def body(*refs):
        if res is not None:
            g_ref, w_ref, b_ref, r_ref, o_ref = refs
        else:
            g_ref, w_ref, b_ref, o_ref = refs
            r_ref = None
        t = pl.program_id(1)
        i = pl.program_id(0)
        acc = _dot(g_ref[...], w_ref[0])

        @pl.when(t == 0)
        def _():
            o_ref[...] = acc

        @pl.when(t > 0)
        def _():
            o_ref[...] = o_ref[...] + acc

        @pl.when(t == T - 1)
        def _():
            v = o_ref[...] + b_ref[0:1, :]
            if r_ref is not None:
                v = v + r_ref[...]
            if relu:
                v = jnp.maximum(v, 0.0)
            rows = i * _BN + lax.broadcasted_iota(jnp.int32, (_BN, cout), 0)
            o_ref[...] = jnp.where(rows < nvalid, v, 0.0)

    return pl.pallas_call(
        body,
        grid=(nb, T),
        in_specs=in_specs,
        out_specs=pl.BlockSpec((_BN, cout), lambda i, t: (i, 0)),
        out_shape=jax.ShapeDtypeStruct((NP, cout), jnp.float32),
        compiler_params=pltpu.CompilerParams(
            dimension_semantics=("parallel", "arbitrary")),
    )(*args)


def _densemm(x, w):
    NP, cin = x.shape
    K = w.shape[1]
    nb = NP // _BN

    def body(x_ref, w_ref, o_ref):
        o_ref[...] = _dot(x_ref[...], w_ref[...])

    return pl.pallas_call(
        body,
        grid=(nb,),
        in_specs=[
            pl.BlockSpec((_BN, cin), lambda i: (i, 0)),
            pl.BlockSpec((cin, K), lambda i: (0, 0)),
        ],
        out_specs=pl.BlockSpec((_BN, K), lambda i: (i, 0)),
        out_shape=jax.ShapeDtypeStruct((NP, K), jnp.float32),
        compiler_params=pltpu.CompilerParams(
            dimension_semantics=("parallel",)),
    )(x, w)


def _skipmm(a, x, wt, wb, bu, bias, nvalid):
    NP, c2 = a.shape
    c = wt.shape[1]
    nb = NP // _BN
    bu8 = jnp.broadcast_to(bu.reshape(1, c2), (8, c2))
    bias8 = jnp.broadcast_to(bias.reshape(1, c), (8, c))

    def body(a_ref, x_ref, wt_ref, wb_ref, bu_ref, bb_ref, o_ref):
        i = pl.program_id(0)
        va = jnp.maximum(a_ref[...] + bu_ref[0:1, :], 0.0)
        v = _dot(va, wt_ref[...]) + _dot(x_ref[...], wb_ref[...]) + bb_ref[0:1, :]
        v = jnp.maximum(v, 0.0)
        rows = i * _BN + lax.broadcasted_iota(jnp.int32, (_BN, c), 0)
        o_ref[...] = jnp.where(rows < nvalid, v, 0.0)

    return pl.pallas_call(
        body,
        grid=(nb,),
        in_specs=[
            pl.BlockSpec((_BN, c2), lambda i: (i, 0)),
            pl.BlockSpec((_BN, c2), lambda i: (i, 0)),
            pl.BlockSpec((c2, c), lambda i: (0, 0)),
            pl.BlockSpec((c2, c), lambda i: (0, 0)),
            pl.BlockSpec((8, c2), lambda i: (0, 0)),
            pl.BlockSpec((8, c), lambda i: (0, 0)),
        ],
        out_specs=pl.BlockSpec((_BN, c), lambda i: (i, 0)),
        out_shape=jax.ShapeDtypeStruct((NP, c), jnp.float32),
        compiler_params=pltpu.CompilerParams(
            dimension_semantics=("parallel",)),
    )(a, x, wt, wb, bu8, bias8)


_BNSCALE = np.float32(np.sqrt(1.0 + 1e-5))


def _conv_w(p, key_w='w', key_g='g'):
    s = p[key_g] / _BNSCALE
    return p[key_w] * s[None, None, :]


def _subm_idx(rb, cap, NP):
    sent = jnp.int32(cap)
    cols = jnp.stack([jnp.where(f, ii.astype(jnp.int32), sent) for ii, f in rb])
    out = jnp.full((len(rb), NP), sent, jnp.int32).at[:, :cap].set(cols)
    return out.reshape(-1)


def _resblock(x, idxflat, p, nvalid):
    g = _gather_rows(x, idxflat)
    h = _tapmm(g, _conv_w(p, 'w1', 'g1'), p['b1'], None, True, nvalid)
    g2 = _gather_rows(h, idxflat)
    return _tapmm(g2, _conv_w(p, 'w2', 'g2'), p['b2'], x, True, nvalid)


def kernel(features, coords, params):
    p = params
    n0 = coords.shape[0]
    cap1 = n0
    cap2 = min(cap1, (D0 // 4) ** 3 + 1)
    cap3 = min(cap2, (D0 // 8) ** 3 + 1)
    NP0 = _pad256(n0)
    NP1 = _pad256(cap1)
    NP2 = _pad256(cap2)
    NP3 = _pad256(cap3)

    rb0 = _subm_rb(coords, D0)
    oc1, inv1, k1 = _down_rb(coords, cap1)
    rb1 = _subm_rb(oc1, D0 // 2)
    oc2, inv2, k2 = _down_rb(oc1, cap2)
    rb2 = _subm_rb(oc2, D0 // 4)
    oc3, inv3, k3 = _down_rb(oc2, cap3)
    rb3 = _subm_rb(oc3, D0 // 8)

    idx0 = _subm_idx(rb0, n0, NP0)
    idx1 = _subm_idx(rb1, cap1, NP1)
    idx2 = _subm_idx(rb2, cap2, NP2)
    idx3 = _subm_idx(rb3, cap3, NP3)

    def down_idx(inv, kidx, n_child, NPout):
        t = jnp.full((8, NPout), jnp.int32(n_child), jnp.int32)
        return t.at[kidx, inv].set(jnp.arange(n_child, dtype=jnp.int32)).reshape(-1)

    idx_d1 = down_idx(inv1, k1, n0, NP1)
    idx_d2 = down_idx(inv2, k2, cap1, NP2)
    idx_d3 = down_idx(inv3, k3, cap2, NP3)

    def up_idx(inv, kidx, n_fine, NPfine, n_coarse):
        t = jnp.full((NPfine,), jnp.int32(n_coarse * 8), jnp.int32)
        v = (inv.astype(jnp.int32) * 8 + kidx.astype(jnp.int32))[:n_fine]
        return t.at[:n_fine].set(v)

    idx_u3 = up_idx(inv3, k3, cap2, NP2, cap3)
    idx_u2 = up_idx(inv2, k2, cap1, NP1, cap2)
    idx_u1 = up_idx(inv1, k1, n0, NP0, cap1)

    def up_w(pp, cin, cout):
        s = pp['g'] / _BNSCALE
        w = pp['w'] * s[None, None, :]
        return jnp.transpose(w, (1, 0, 2)).reshape(cin, 8 * cout)

    f16 = jnp.zeros((NP0, 16), jnp.float32).at[:n0, :6].set(features)

    w_in = jnp.pad(_conv_w(p['input_conv']), ((0, 0), (0, 10), (0, 0)))
    g = _gather_rows(f16, idx0)
    x0 = _tapmm(g, w_in, p['input_conv']['b'], None, True, n0)
    x1 = _resblock(x0, idx0, p['enc1'], n0)

    g = _gather_rows(x1, idx_d1)
    x2 = _tapmm(g, _conv_w(p['down1']), p['down1']['b'], None, True, cap1)
    x2 = _resblock(x2, idx1, p['enc2'], cap1)

    g = _gather_rows(x2, idx_d2)
    x3 = _tapmm(g, _conv_w(p['down2']), p['down2']['b'], None, True, cap2)
    x3 = _resblock(x3, idx2, p['enc3'], cap2)

    g = _gather_rows(x3, idx_d3)
    x4 = _tapmm(g, _conv_w(p['down3']), p['down3']['b'], None, True, cap3)
    x4 = _resblock(x4, idx3, p['bottleneck'], cap3)

    P = _densemm(x4, up_w(p['up3'], 256, 128)).reshape(NP3 * 8, 128)
    gu = _gather_rows(P, idx_u3)
    d3 = _skipmm(gu, x3, p['skip3']['w'][:128], p['skip3']['w'][128:],
                 p['up3']['b'], p['skip3']['b'], cap2)
    d3 = _resblock(d3, idx2, p['dec3'], cap2)

    P = _densemm(d3, up_w(p['up2'], 128, 64)).reshape(NP2 * 8, 64)
    gu = _gather_rows(P, idx_u2)
    d2 = _skipmm(gu, x2, p['skip2']['w'][:64], p['skip2']['w'][64:],
                 p['up2']['b'], p['skip2']['b'], cap1)
    d2 = _resblock(d2, idx1, p['dec2'], cap1)

    P = _densemm(d2, up_w(p['up1'], 64, 32)).reshape(NP1 * 8, 32)
    gu = _gather_rows(P, idx_u1)
    d1 = _skipmm(gu, x1, p['skip1']['w'][:32], p['skip1']['w'][32:],
                 p['up1']['b'], p['skip1']['b'], n0)
    d1 = _resblock(d1, idx0, p['dec1'], n0)

    return d1[:n0]
